# baseline (speedup 1.0000x reference)
import functools

import jax
import jax.numpy as jnp
from jax import lax
from jax.experimental import pallas as pl
from jax.experimental.pallas import tpu as pltpu
from jax.experimental.pallas import tpu_sc as plsc

_B, _C, _S, _H = 1024, 26, 50, 32
_M = _B * _C * _S
_NC, _NS = 2, 16
_NW = _NC * _NS
_GROUP = 128
_NGROUPS = _M // _GROUP
_GPC = 8
_NCHUNKS = 41
_GROUPS_PER_W = _GPC * _NCHUNKS
_NGROUPS_PAD = _NW * _GROUPS_PER_W
_M_PAD = _NGROUPS_PAD * _GROUP
_CHUNK = _GPC * _GROUP
_B4 = _B // 4


def _sc_gather(idx2d, word_table):
    mesh = plsc.VectorSubcoreMesh(core_axis_name="c", subcore_axis_name="s")

    @functools.partial(
        pl.kernel,
        mesh=mesh,
        out_type=jax.ShapeDtypeStruct((_M_PAD, _H), jnp.bfloat16),
        compiler_params=pltpu.CompilerParams(use_tc_tiling_on_sc=False),
        scratch_types=[
            pltpu.VMEM((_GPC, _GROUP), jnp.int32),
            pltpu.VMEM((_CHUNK, _H), jnp.bfloat16),
            pltpu.SemaphoreType.DMA,
        ],
    )
    def k(idx_hbm, table_hbm, out_hbm, idx_v, rows_v, gsem):
        wid = lax.axis_index("s") * _NC + lax.axis_index("c")
        g_base = wid * _GROUPS_PER_W

        @pl.loop(0, _NCHUNKS)
        def _(i):
            g0 = pl.multiple_of(g_base + i * _GPC, 8)
            pltpu.sync_copy(idx_hbm.at[pl.ds(g0, _GPC)], idx_v)
            copies = []
            for j in range(_GPC):
                copies.append(
                    pltpu.async_copy(
                        table_hbm.at[idx_v.at[j]],
                        rows_v.at[pl.ds(j * _GROUP, _GROUP)],
                        gsem,
                    )
                )
            for cp in copies:
                cp.wait()
            row0 = pl.multiple_of(g0 * _GROUP, 1024)
            pltpu.sync_copy(rows_v, out_hbm.at[pl.ds(row0, _CHUNK)])

    return k(idx2d, word_table)


def _tc_body(w_ref, weg_ref, bias_ref, gamma_ref, beta_ref, G_ref, o_ref):
    s = pl.program_id(0)
    c = pl.program_id(1)
    X = weg_ref[...].astype(jnp.float32)
    emb = X + bias_ref[0, 0]
    G = G_ref[...]
    s1 = jax.lax.dot(emb, G, precision=jax.lax.Precision.HIGHEST)
    s2 = jax.lax.dot(emb * emb, G, precision=jax.lax.Precision.HIGHEST)
    mu = s1 * (1.0 / _H)
    var = s2 * (1.0 / _H) - mu * mu
    rstd = jax.lax.rsqrt(var + 1e-5)
    wsc = w_ref[s, c]
    a = rstd * wsc
    contrib = (emb - mu) * (a * gamma_ref[...]) + wsc * beta_ref[...]

    @pl.when(c == 0)
    def _():
        o_ref[0] = contrib

    @pl.when(c > 0)
    def _():
        o_ref[0] += contrib


def _tc_fuse(weg2, w, bias, gamma128, beta128, G):
    return pl.pallas_call(
        _tc_body,
        grid=(_S, _C),
        in_specs=[
            pl.BlockSpec(memory_space=pltpu.SMEM),
            pl.BlockSpec((_B4, 128), lambda s, c: (c * _S + s, 0)),
            pl.BlockSpec((1, 1, 1, 128), lambda s, c: (c, s, 0, 0)),
            pl.BlockSpec((1, 128), lambda s, c: (0, 0)),
            pl.BlockSpec((1, 128), lambda s, c: (0, 0)),
            pl.BlockSpec((128, 128), lambda s, c: (0, 0)),
        ],
        out_specs=pl.BlockSpec((1, _B4, 128), lambda s, c: (s, 0, 0)),
        out_shape=jax.ShapeDtypeStruct((_S, _B4, 128), jnp.float32),
    )(w, weg2, bias, gamma128, beta128, G)


def kernel(x, word_table, pos_table, ch_table, ln_gamma, ln_beta, fusion_w):
    idx = jnp.transpose(x, (1, 2, 0)).reshape(_M)
    idx = jnp.concatenate([idx, jnp.zeros((_M_PAD - _M,), jnp.int32)])
    idx2d = idx.reshape(_NGROUPS_PAD, _GROUP)
    weg = _sc_gather(idx2d, word_table.astype(jnp.bfloat16))
    weg2 = weg.reshape(_M_PAD // 4, 128)

    w = jax.nn.softmax(fusion_w, axis=-1)
    bias = ch_table[:, None, :] + pos_table[None, :, :]
    bias = jnp.tile(bias, (1, 1, 4)).reshape(_C, _S, 1, 128)
    gamma128 = jnp.tile(ln_gamma, 4).reshape(1, 128)
    beta128 = jnp.tile(ln_beta, 4).reshape(1, 128)
    gi = jax.lax.broadcasted_iota(jnp.int32, (128, 128), 0) // _H
    gj = jax.lax.broadcasted_iota(jnp.int32, (128, 128), 1) // _H
    G = (gi == gj).astype(jnp.float32)

    out4 = _tc_fuse(weg2, w, bias, gamma128, beta128, G)
    return jnp.transpose(out4.reshape(_S, _B, _H), (1, 0, 2))


# speedup vs baseline: 1.4560x; 1.4560x over previous
import functools

import jax
import jax.numpy as jnp
from jax import lax
from jax.experimental import pallas as pl
from jax.experimental.pallas import tpu as pltpu
from jax.experimental.pallas import tpu_sc as plsc

_B, _C, _S, _H = 1024, 26, 50, 32
_M = _B * _C * _S
_NC, _NS = 2, 16
_NW = _NC * _NS
_GROUP = 128
_GPC = 8
_CHUNK = _GPC * _GROUP
_NPAIRS = _C * _S
_B4 = _B // 4
_KSLICES = 2
_CSLICE = _C // _KSLICES


def _sc_gather(idx2d, word_table, pair_lo, npairs):
    mesh = plsc.VectorSubcoreMesh(core_axis_name="c", subcore_axis_name="s")

    @functools.partial(
        pl.kernel,
        mesh=mesh,
        out_type=jax.ShapeDtypeStruct((npairs * _CHUNK, _H), jnp.float32),
        compiler_params=pltpu.CompilerParams(use_tc_tiling_on_sc=False),
        scratch_types=[
            pltpu.VMEM((_GPC, _GROUP), jnp.int32),
            pltpu.VMEM((_CHUNK, _H), jnp.float32),
            pltpu.SemaphoreType.DMA,
        ],
    )
    def k(idx_hbm, table_hbm, out_hbm, idx_v, rows_v, gsem):
        wid = lax.axis_index("s") * _NC + lax.axis_index("c")
        p_start = (npairs * wid) // _NW
        p_end = (npairs * (wid + 1)) // _NW

        @pl.loop(0, p_end - p_start)
        def _(i):
            p = p_start + i
            g0 = pl.multiple_of((pair_lo + p) * _GPC, 8)
            pltpu.sync_copy(idx_hbm.at[pl.ds(g0, _GPC)], idx_v)
            copies = []
            for j in range(_GPC):
                copies.append(
                    pltpu.async_copy(
                        table_hbm.at[idx_v.at[j]],
                        rows_v.at[pl.ds(j * _GROUP, _GROUP)],
                        gsem,
                    )
                )
            for cp in copies:
                cp.wait()
            row0 = pl.multiple_of(p * _CHUNK, 1024)
            pltpu.sync_copy(rows_v, out_hbm.at[pl.ds(row0, _CHUNK)])

    return k(idx2d, word_table)


def _tc_body(w_ref, weg_ref, bias_ref, gamma_ref, beta_ref, G_ref, o_ref):
    s = pl.program_id(0)
    c = pl.program_id(1)
    X = weg_ref[...]
    emb = X + bias_ref[0, 0]
    G = G_ref[...]
    s1 = jax.lax.dot(emb, G)
    s2 = jax.lax.dot(emb * emb, G)
    mu = s1 * (1.0 / _H)
    var = s2 * (1.0 / _H) - mu * mu
    rstd = jax.lax.rsqrt(var + 1e-5)
    wsc = w_ref[s, c]
    a = rstd * wsc
    contrib = (emb - mu) * (a * gamma_ref[...]) + wsc * beta_ref[...]

    @pl.when(c == 0)
    def _():
        o_ref[0] = contrib

    @pl.when(c > 0)
    def _():
        o_ref[0] += contrib


def _tc_fuse(weg2, w, bias, gamma128, beta128, G, ncs):
    return pl.pallas_call(
        _tc_body,
        grid=(_S, ncs),
        in_specs=[
            pl.BlockSpec(memory_space=pltpu.SMEM),
            pl.BlockSpec((_B4, 128), lambda s, c: (c * _S + s, 0)),
            pl.BlockSpec((1, 1, 1, 128), lambda s, c: (c, s, 0, 0)),
            pl.BlockSpec((1, 128), lambda s, c: (0, 0)),
            pl.BlockSpec((1, 128), lambda s, c: (0, 0)),
            pl.BlockSpec((128, 128), lambda s, c: (0, 0)),
        ],
        out_specs=pl.BlockSpec((1, _B4, 128), lambda s, c: (s, 0, 0)),
        out_shape=jax.ShapeDtypeStruct((_S, _B4, 128), jnp.float32),
    )(w, weg2, bias, gamma128, beta128, G)


def kernel(x, word_table, pos_table, ch_table, ln_gamma, ln_beta, fusion_w):
    idx2d = jnp.transpose(x, (1, 2, 0)).reshape(_NPAIRS * _GPC, _GROUP)

    w = jax.nn.softmax(fusion_w, axis=-1)
    bias = ch_table[:, None, :] + pos_table[None, :, :]
    bias = jnp.tile(bias, (1, 1, 4)).reshape(_C, _S, 1, 128)
    gamma128 = jnp.tile(ln_gamma, 4).reshape(1, 128)
    beta128 = jnp.tile(ln_beta, 4).reshape(1, 128)
    gi = jax.lax.broadcasted_iota(jnp.int32, (128, 128), 0) // _H
    gj = jax.lax.broadcasted_iota(jnp.int32, (128, 128), 1) // _H
    G = (gi == gj).astype(jnp.float32)

    out4 = None
    npairs = _CSLICE * _S
    for k in range(_KSLICES):
        c_lo = k * _CSLICE
        weg = _sc_gather(idx2d, word_table, c_lo * _S, npairs)
        weg2 = weg.reshape(npairs * _CHUNK // 4, 128)
        part = _tc_fuse(weg2, w[:, c_lo:c_lo + _CSLICE],
                        bias[c_lo:c_lo + _CSLICE], gamma128, beta128, G,
                        _CSLICE)
        out4 = part if out4 is None else out4 + part
    return jnp.transpose(out4.reshape(_S, _B, _H), (1, 0, 2))


# speedup vs baseline: 2.6229x; 1.8015x over previous
import functools

import jax
import jax.numpy as jnp
from jax import lax
from jax.experimental import pallas as pl
from jax.experimental.pallas import tpu as pltpu
from jax.experimental.pallas import tpu_sc as plsc

_B, _C, _S, _H = 1024, 26, 50, 32
_M = _B * _C * _S
_NC, _NS = 2, 16
_NW = _NC * _NS
_GROUP = 128
_GPC = 8
_CHUNK = _GPC * _GROUP
_NPAIRS = _C * _S
_B4 = _B // 4
_KSLICES = 2
_CSLICE = _C // _KSLICES


def _sc_gather(idx2d, word_table, pair_lo, npairs):
    mesh = plsc.VectorSubcoreMesh(core_axis_name="c", subcore_axis_name="s")

    @functools.partial(
        pl.kernel,
        mesh=mesh,
        out_type=jax.ShapeDtypeStruct((npairs * _CHUNK, _H), jnp.float32),
        compiler_params=pltpu.CompilerParams(use_tc_tiling_on_sc=False),
        scratch_types=[
            pltpu.VMEM((_GPC, _GROUP), jnp.int32),
            pltpu.VMEM((_CHUNK, _H), jnp.float32),
            pltpu.SemaphoreType.DMA,
        ],
    )
    def k(idx_hbm, table_hbm, out_hbm, idx_v, rows_v, gsem):
        wid = lax.axis_index("s") * _NC + lax.axis_index("c")
        p_start = (npairs * wid) // _NW
        p_end = (npairs * (wid + 1)) // _NW

        @pl.loop(0, p_end - p_start)
        def _(i):
            p = p_start + i
            g0 = pl.multiple_of((pair_lo + p) * _GPC, 8)
            pltpu.sync_copy(idx_hbm.at[pl.ds(g0, _GPC)], idx_v)
            copies = []
            for j in range(_GPC):
                copies.append(
                    pltpu.async_copy(
                        table_hbm.at[idx_v.at[j]],
                        rows_v.at[pl.ds(j * _GROUP, _GROUP)],
                        gsem,
                    )
                )
            for cp in copies:
                cp.wait()
            row0 = pl.multiple_of(p * _CHUNK, 1024)
            pltpu.sync_copy(rows_v, out_hbm.at[pl.ds(row0, _CHUNK)])

    return k(idx2d, word_table)


_NSB = 25
_RB = _NSB * _B4


def _tc_body(weg_ref, bias_ref, wrow_ref, gamma_ref, beta_ref, G_ref, o_ref):
    c = pl.program_id(1)
    X = weg_ref[...]
    bexp = jnp.broadcast_to(
        bias_ref[0, :, 0, :][:, None, :], (_NSB, _B4, 128)).reshape(_RB, 128)
    wexp = jnp.broadcast_to(
        wrow_ref[0, :, 0, :][:, None, :], (_NSB, _B4, 128)).reshape(_RB, 128)
    emb = X + bexp
    G = G_ref[...]
    s1 = jax.lax.dot(emb, G)
    s2 = jax.lax.dot(emb * emb, G)
    mu = s1 * (1.0 / _H)
    var = s2 * (1.0 / _H) - mu * mu
    rstd = jax.lax.rsqrt(var + 1e-5)
    contrib = (emb - mu) * (rstd * wexp * gamma_ref[...]) \
        + wexp * beta_ref[...]
    contrib = contrib.reshape(_NSB, _B4, 128)

    @pl.when(c == 0)
    def _():
        o_ref[...] = contrib

    @pl.when(c > 0)
    def _():
        o_ref[...] += contrib


def _tc_fuse(weg2, wrow, bias, gamma128, beta128, G, ncs):
    return pl.pallas_call(
        _tc_body,
        grid=(_S // _NSB, ncs),
        in_specs=[
            pl.BlockSpec((_RB, 128),
                         lambda sb, c: (c * (_S // _NSB) + sb, 0)),
            pl.BlockSpec((1, _NSB, 1, 128), lambda sb, c: (c, sb, 0, 0)),
            pl.BlockSpec((1, _NSB, 1, 128), lambda sb, c: (c, sb, 0, 0)),
            pl.BlockSpec((1, 128), lambda sb, c: (0, 0)),
            pl.BlockSpec((1, 128), lambda sb, c: (0, 0)),
            pl.BlockSpec((128, 128), lambda sb, c: (0, 0)),
        ],
        out_specs=pl.BlockSpec((_NSB, _B4, 128), lambda sb, c: (sb, 0, 0)),
        out_shape=jax.ShapeDtypeStruct((_S, _B4, 128), jnp.float32),
    )(weg2, bias, wrow, gamma128, beta128, G)


def kernel(x, word_table, pos_table, ch_table, ln_gamma, ln_beta, fusion_w):
    idx2d = jnp.transpose(x, (1, 2, 0)).reshape(_NPAIRS * _GPC, _GROUP)

    w = jax.nn.softmax(fusion_w, axis=-1)
    wrow = jnp.broadcast_to(w.T[:, :, None], (_C, _S, 128))
    wrow = wrow.reshape(_C, _S, 1, 128)
    bias = ch_table[:, None, :] + pos_table[None, :, :]
    bias = jnp.tile(bias, (1, 1, 4)).reshape(_C, _S, 1, 128)
    gamma128 = jnp.tile(ln_gamma, 4).reshape(1, 128)
    beta128 = jnp.tile(ln_beta, 4).reshape(1, 128)
    gi = jax.lax.broadcasted_iota(jnp.int32, (128, 128), 0) // _H
    gj = jax.lax.broadcasted_iota(jnp.int32, (128, 128), 1) // _H
    G = (gi == gj).astype(jnp.float32)

    out4 = None
    npairs = _CSLICE * _S
    for k in range(_KSLICES):
        c_lo = k * _CSLICE
        weg = _sc_gather(idx2d, word_table, c_lo * _S, npairs)
        weg2 = weg.reshape(npairs * _CHUNK // 4, 128)
        part = _tc_fuse(weg2, wrow[c_lo:c_lo + _CSLICE],
                        bias[c_lo:c_lo + _CSLICE], gamma128, beta128, G,
                        _CSLICE)
        out4 = part if out4 is None else out4 + part
    return jnp.transpose(out4.reshape(_S, _B, _H), (1, 0, 2))
